# baseline (speedup 1.0000x reference)
import functools
import math

import jax
import jax.numpy as jnp
from jax import lax
from jax.experimental import pallas as pl
from jax.experimental.pallas import tpu as pltpu
from jax.experimental.pallas import tpu_sc as plsc

D = 768
E = 8
H = 192
N = 8192
T = 384
NB = -(-N // T) + E
CAP = NB * T
NW = 32
TPW = N // NW
CHUNK = 128
NCH = TPW // CHUNK


def _router_body(x_ref, wr_ref, br_ref, eid_ref):
    logits = lax.dot_general(
        x_ref[...].astype(jnp.bfloat16), wr_ref[...].astype(jnp.bfloat16),
        (((1,), (0,)), ((), ())),
        preferred_element_type=jnp.float32) + br_ref[...]
    mx = jnp.max(logits, axis=1, keepdims=True)
    cand = jnp.where(logits == mx,
                     lax.broadcasted_iota(jnp.int32, logits.shape, 1), E)
    eid_ref[...] = jnp.min(cand, axis=1, keepdims=True)


def _router(xf, Wr, br):
    tb = 2048
    return pl.pallas_call(
        _router_body,
        grid=(N // tb,),
        in_specs=[
            pl.BlockSpec((tb, D), lambda i: (i, 0)),
            pl.BlockSpec((D, E), lambda i: (0, 0)),
            pl.BlockSpec((1, E), lambda i: (0, 0)),
        ],
        out_specs=pl.BlockSpec((tb, 1), lambda i: (i, 0)),
        out_shape=jax.ShapeDtypeStruct((N, 1), jnp.int32),
    )(xf, Wr, br.reshape(1, E))


def _rank_body(eid_ref, pos_ref, blk_ref):
    eid = eid_ref[...]
    rows, cols = eid.shape
    ri = lax.broadcasted_iota(jnp.int32, (cols, cols), 0)
    ci = lax.broadcasted_iota(jnp.int32, (cols, cols), 1)
    lt_incl = (ri <= ci).astype(jnp.float32)
    r2 = lax.broadcasted_iota(jnp.int32, (rows, rows), 0)
    c2 = lax.broadcasted_iota(jnp.int32, (rows, rows), 1)
    lt_excl = (c2 < r2).astype(jnp.float32)

    def mm(a, b):
        return lax.dot_general(a, b, (((1,), (0,)), ((), ())),
                               preferred_element_type=jnp.float32)

    pos = jnp.zeros((rows, cols), jnp.float32)
    off = jnp.zeros((1, 1), jnp.float32)
    ends = []
    for e in range(E):
        m = (eid == e).astype(jnp.float32)
        rc = mm(m, lt_incl)
        rt = jnp.sum(m, axis=1, keepdims=True)
        co = mm(lt_excl, rt)
        rank = co + rc - m
        pos = pos + m * (off + rank)
        cnt = jnp.sum(rt, axis=0, keepdims=True)
        off = off + jnp.ceil(cnt / T) * T
        ends.append(off)
    pos_ref[...] = pos.astype(jnp.int32)

    ends_cat = jnp.concatenate(ends, axis=1)
    bstart = (lax.broadcasted_iota(jnp.int32, (NB, 1), 0) * T).astype(jnp.float32)
    be = jnp.sum((ends_cat <= bstart).astype(jnp.int32), axis=1, keepdims=True)
    blk_ref[...] = jnp.where(bstart < off, jnp.minimum(be, E - 1), -1)


def _rank(eid2d):
    return pl.pallas_call(
        _rank_body,
        out_shape=(jax.ShapeDtypeStruct((N // 128, 128), jnp.int32),
                   jax.ShapeDtypeStruct((NB, 1), jnp.int32)),
    )(eid2d)


def _make_sc_dispatch():
    mesh = plsc.VectorSubcoreMesh(core_axis_name="c", subcore_axis_name="s",
                                  num_cores=2, num_subcores=16)

    @functools.partial(
        pl.kernel,
        out_type=jax.ShapeDtypeStruct((CAP, D), jnp.float32),
        mesh=mesh,
        scratch_types=[
            [pltpu.VMEM((CHUNK,), jnp.int32) for _ in range(NCH)],
            pltpu.VMEM((CHUNK, D), jnp.float32),
            pltpu.SemaphoreType.DMA,
        ],
    )
    def dispatch(pos_hbm, x_hbm, xs_hbm, idxs, rows_v, sem):
        wid = lax.axis_index("s") * 2 + lax.axis_index("c")
        base = wid * TPW
        for k in range(NCH):
            cb = base + k * CHUNK
            pltpu.sync_copy(pos_hbm.at[pl.ds(cb, CHUNK)], idxs[k])
            pltpu.sync_copy(x_hbm.at[pl.ds(cb, CHUNK)], rows_v)
            pltpu.async_copy(rows_v, xs_hbm.at[idxs[k]], sem).wait()

    return dispatch


def _expert_body(be_ref, xs_ref, w1_ref, b1_ref, w2_ref, b2_ref, ys_ref):
    e = be_ref[pl.program_id(0)]

    @pl.when(e >= 0)
    def _():
        w1 = w1_ref[e]
        w2 = w2_ref[e]
        b1 = b1_ref[e]
        b2 = b2_ref[e]
        ST = T // 4
        for s in range(T // ST):
            xb = xs_ref[pl.ds(s * ST, ST), :]
            h = jnp.dot(xb.astype(jnp.bfloat16), w1,
                        preferred_element_type=jnp.float32) + b1
            g = h * 0.5 * (1.0 + lax.erf(h * (1.0 / math.sqrt(2.0))))
            y = jnp.dot(g.astype(jnp.bfloat16), w2,
                        preferred_element_type=jnp.float32)
            ys_ref[pl.ds(s * ST, ST), :] = xb + y + b2


def _experts(blk_exp, xs, W1, b1, W2, b2):
    grid_spec = pltpu.PrefetchScalarGridSpec(
        num_scalar_prefetch=1,
        grid=(NB,),
        in_specs=[
            pl.BlockSpec((T, D), lambda i, be: (i, 0)),
            pl.BlockSpec((E, D, H), lambda i, be: (0, 0, 0)),
            pl.BlockSpec((E, 1, H), lambda i, be: (0, 0, 0)),
            pl.BlockSpec((E, H, D), lambda i, be: (0, 0, 0)),
            pl.BlockSpec((E, 1, D), lambda i, be: (0, 0, 0)),
        ],
        out_specs=pl.BlockSpec((T, D), lambda i, be: (i, 0)),
    )
    return pl.pallas_call(
        _expert_body,
        grid_spec=grid_spec,
        out_shape=jax.ShapeDtypeStruct((CAP, D), jnp.float32),
    )(blk_exp, xs, W1.astype(jnp.bfloat16), b1.reshape(E, 1, H),
      W2.astype(jnp.bfloat16), b2.reshape(E, 1, D))


def _make_sc_combine():
    mesh = plsc.VectorSubcoreMesh(core_axis_name="c", subcore_axis_name="s",
                                  num_cores=2, num_subcores=16)

    @functools.partial(
        pl.kernel,
        out_type=jax.ShapeDtypeStruct((N, D), jnp.float32),
        mesh=mesh,
        scratch_types=[
            [pltpu.VMEM((CHUNK,), jnp.int32) for _ in range(NCH)],
            pltpu.VMEM((CHUNK, D), jnp.float32),
            pltpu.SemaphoreType.DMA,
        ],
    )
    def combine(pos_hbm, ys_hbm, out_hbm, idxs, rows_v, sem):
        wid = lax.axis_index("s") * 2 + lax.axis_index("c")
        base = wid * TPW
        for k in range(NCH):
            cb = base + k * CHUNK
            pltpu.sync_copy(pos_hbm.at[pl.ds(cb, CHUNK)], idxs[k])
            pltpu.async_copy(ys_hbm.at[idxs[k]], rows_v, sem).wait()
            pltpu.sync_copy(rows_v, out_hbm.at[pl.ds(cb, CHUNK)])

    return combine


_sc_dispatch = functools.cache(_make_sc_dispatch)
_sc_combine = functools.cache(_make_sc_combine)


def kernel(x, Wr, br, W1, b1, W2, b2):
    B, S, _ = x.shape
    xf = x.reshape(N, D)
    eid = _router(xf, Wr, br)
    pos2d, blk = _rank(eid.reshape(N // 128, 128))
    pos = pos2d.reshape(N)
    xs = _sc_dispatch()(pos, xf)
    ys = _experts(blk.reshape(NB), xs, W1, b1, W2, b2)
    outf = _sc_combine()(pos, ys)
    return outf.reshape(B, S, D)


# speedup vs baseline: 1.0781x; 1.0781x over previous
import functools
import math

import jax
import jax.numpy as jnp
from jax import lax
from jax.experimental import pallas as pl
from jax.experimental.pallas import tpu as pltpu
from jax.experimental.pallas import tpu_sc as plsc

D = 768
E = 8
H = 192
N = 8192
T = 512
NB = -(-N // T) + E
CAP = NB * T
NW = 32
TPW = N // NW
CHUNK = 128
NCH = TPW // CHUNK


def _router_body(x_ref, wr_ref, br_ref, eid_ref):
    logits = lax.dot_general(
        x_ref[...].astype(jnp.bfloat16), wr_ref[...].astype(jnp.bfloat16),
        (((1,), (0,)), ((), ())),
        preferred_element_type=jnp.float32) + br_ref[...]
    mx = jnp.max(logits, axis=1, keepdims=True)
    cand = jnp.where(logits == mx,
                     lax.broadcasted_iota(jnp.int32, logits.shape, 1), E)
    eid_ref[...] = jnp.min(cand, axis=1, keepdims=True)


def _router(xf, Wr, br):
    tb = 2048
    return pl.pallas_call(
        _router_body,
        grid=(N // tb,),
        in_specs=[
            pl.BlockSpec((tb, D), lambda i: (i, 0)),
            pl.BlockSpec((D, E), lambda i: (0, 0)),
            pl.BlockSpec((1, E), lambda i: (0, 0)),
        ],
        out_specs=pl.BlockSpec((tb, 1), lambda i: (i, 0)),
        out_shape=jax.ShapeDtypeStruct((N, 1), jnp.int32),
    )(xf, Wr, br.reshape(1, E))


def _rank_body(eid_ref, pos_ref, blk_ref):
    eid = eid_ref[...]
    rows, cols = eid.shape
    ri = lax.broadcasted_iota(jnp.int32, (cols, cols), 0)
    ci = lax.broadcasted_iota(jnp.int32, (cols, cols), 1)
    lt_incl = (ri <= ci).astype(jnp.float32)
    r2 = lax.broadcasted_iota(jnp.int32, (rows, rows), 0)
    c2 = lax.broadcasted_iota(jnp.int32, (rows, rows), 1)
    lt_excl = (c2 < r2).astype(jnp.float32)

    def mm(a, b):
        return lax.dot_general(a, b, (((1,), (0,)), ((), ())),
                               preferred_element_type=jnp.float32)

    pos = jnp.zeros((rows, cols), jnp.float32)
    off = jnp.zeros((1, 1), jnp.float32)
    ends = []
    for e in range(E):
        m = (eid == e).astype(jnp.float32)
        rc = mm(m, lt_incl)
        rt = jnp.sum(m, axis=1, keepdims=True)
        co = mm(lt_excl, rt)
        rank = co + rc - m
        pos = pos + m * (off + rank)
        cnt = jnp.sum(rt, axis=0, keepdims=True)
        off = off + jnp.ceil(cnt / T) * T
        ends.append(off)
    pos_ref[...] = pos.astype(jnp.int32)

    ends_cat = jnp.concatenate(ends, axis=1)
    bstart = (lax.broadcasted_iota(jnp.int32, (NB, 1), 0) * T).astype(jnp.float32)
    be = jnp.sum((ends_cat <= bstart).astype(jnp.int32), axis=1, keepdims=True)
    blk_ref[...] = jnp.where(bstart < off, jnp.minimum(be, E - 1), -1)


def _rank(eid2d):
    return pl.pallas_call(
        _rank_body,
        out_shape=(jax.ShapeDtypeStruct((N // 128, 128), jnp.int32),
                   jax.ShapeDtypeStruct((NB, 1), jnp.int32)),
    )(eid2d)


def _make_sc_dispatch():
    mesh = plsc.VectorSubcoreMesh(core_axis_name="c", subcore_axis_name="s",
                                  num_cores=2, num_subcores=16)

    @functools.partial(
        pl.kernel,
        out_type=jax.ShapeDtypeStruct((CAP, D), jnp.float32),
        mesh=mesh,
        scratch_types=[
            [pltpu.VMEM((CHUNK,), jnp.int32) for _ in range(NCH)],
            pltpu.VMEM((CHUNK, D), jnp.float32),
            pltpu.SemaphoreType.DMA,
        ],
    )
    def dispatch(pos_hbm, x_hbm, xs_hbm, idxs, rows_v, sem):
        wid = lax.axis_index("s") * 2 + lax.axis_index("c")
        base = wid * TPW
        for k in range(NCH):
            cb = base + k * CHUNK
            pltpu.sync_copy(pos_hbm.at[pl.ds(cb, CHUNK)], idxs[k])
            pltpu.sync_copy(x_hbm.at[pl.ds(cb, CHUNK)], rows_v)
            pltpu.async_copy(rows_v, xs_hbm.at[idxs[k]], sem).wait()

    return dispatch


def _expert_body(be_ref, xs_ref, w1_ref, b1_ref, w2_ref, b2_ref, ys_ref):
    e = be_ref[pl.program_id(0)]

    @pl.when(e >= 0)
    def _():
        w1 = w1_ref[e]
        w2 = w2_ref[e]
        b1 = b1_ref[e]
        b2 = b2_ref[e]
        ST = T // 4
        for s in range(T // ST):
            xb = xs_ref[pl.ds(s * ST, ST), :]
            h = jnp.dot(xb.astype(jnp.bfloat16), w1,
                        preferred_element_type=jnp.float32) + b1
            g = h * 0.5 * (1.0 + lax.erf(h * (1.0 / math.sqrt(2.0))))
            y = jnp.dot(g.astype(jnp.bfloat16), w2,
                        preferred_element_type=jnp.float32)
            ys_ref[pl.ds(s * ST, ST), :] = xb + y + b2


def _experts(blk_exp, xs, W1, b1, W2, b2):
    grid_spec = pltpu.PrefetchScalarGridSpec(
        num_scalar_prefetch=1,
        grid=(NB,),
        in_specs=[
            pl.BlockSpec((T, D), lambda i, be: (i, 0)),
            pl.BlockSpec((E, D, H), lambda i, be: (0, 0, 0)),
            pl.BlockSpec((E, 1, H), lambda i, be: (0, 0, 0)),
            pl.BlockSpec((E, H, D), lambda i, be: (0, 0, 0)),
            pl.BlockSpec((E, 1, D), lambda i, be: (0, 0, 0)),
        ],
        out_specs=pl.BlockSpec((T, D), lambda i, be: (i, 0)),
    )
    return pl.pallas_call(
        _expert_body,
        grid_spec=grid_spec,
        out_shape=jax.ShapeDtypeStruct((CAP, D), jnp.float32),
    )(blk_exp, xs, W1.astype(jnp.bfloat16), b1.reshape(E, 1, H),
      W2.astype(jnp.bfloat16), b2.reshape(E, 1, D))


def _make_sc_combine():
    mesh = plsc.VectorSubcoreMesh(core_axis_name="c", subcore_axis_name="s",
                                  num_cores=2, num_subcores=16)

    @functools.partial(
        pl.kernel,
        out_type=jax.ShapeDtypeStruct((N, D), jnp.float32),
        mesh=mesh,
        scratch_types=[
            [pltpu.VMEM((CHUNK,), jnp.int32) for _ in range(NCH)],
            pltpu.VMEM((CHUNK, D), jnp.float32),
            pltpu.SemaphoreType.DMA,
        ],
    )
    def combine(pos_hbm, ys_hbm, out_hbm, idxs, rows_v, sem):
        wid = lax.axis_index("s") * 2 + lax.axis_index("c")
        base = wid * TPW
        for k in range(NCH):
            cb = base + k * CHUNK
            pltpu.sync_copy(pos_hbm.at[pl.ds(cb, CHUNK)], idxs[k])
            pltpu.async_copy(ys_hbm.at[idxs[k]], rows_v, sem).wait()
            pltpu.sync_copy(rows_v, out_hbm.at[pl.ds(cb, CHUNK)])

    return combine


_sc_dispatch = functools.cache(_make_sc_dispatch)
_sc_combine = functools.cache(_make_sc_combine)


def kernel(x, Wr, br, W1, b1, W2, b2):
    B, S, _ = x.shape
    xf = x.reshape(N, D)
    eid = _router(xf, Wr, br)
    pos2d, blk = _rank(eid.reshape(N // 128, 128))
    pos = pos2d.reshape(N)
    xs = _sc_dispatch()(pos, xf)
    ys = _experts(blk.reshape(NB), xs, W1, b1, W2, b2)
    outf = _sc_combine()(pos, ys)
    return outf.reshape(B, S, D)


# speedup vs baseline: 1.1279x; 1.0462x over previous
import functools
import math

import jax
import jax.numpy as jnp
from jax import lax
from jax.experimental import pallas as pl
from jax.experimental.pallas import tpu as pltpu
from jax.experimental.pallas import tpu_sc as plsc

D = 768
E = 8
H = 192
N = 8192
T = 512
NB = -(-N // T) + E
CAP = NB * T
NW = 32
TPW = N // NW
CHUNK = 128
NCH = TPW // CHUNK


def _router_body(x_ref, wr_ref, br_ref, eid_ref):
    logits = lax.dot_general(
        x_ref[...].astype(jnp.bfloat16), wr_ref[...].astype(jnp.bfloat16),
        (((1,), (0,)), ((), ())),
        preferred_element_type=jnp.float32) + br_ref[...]
    mx = jnp.max(logits, axis=1, keepdims=True)
    cand = jnp.where(logits == mx,
                     lax.broadcasted_iota(jnp.int32, logits.shape, 1), E)
    am = jnp.min(cand, axis=1)
    eid_ref[...] = am.reshape(eid_ref.shape)


def _router(xf, Wr, br):
    tb = 2048
    return pl.pallas_call(
        _router_body,
        grid=(N // tb,),
        in_specs=[
            pl.BlockSpec((tb, D), lambda i: (i, 0)),
            pl.BlockSpec((D, E), lambda i: (0, 0)),
            pl.BlockSpec((1, E), lambda i: (0, 0)),
        ],
        out_specs=pl.BlockSpec((tb // 128, 128), lambda i: (i, 0)),
        out_shape=jax.ShapeDtypeStruct((N // 128, 128), jnp.int32),
    )(xf, Wr, br.reshape(1, E))


def _rank_body(eid_ref, pos_ref, blk_ref):
    eid = eid_ref[...]
    rows, cols = eid.shape
    ri = lax.broadcasted_iota(jnp.int32, (cols, cols), 0)
    ci = lax.broadcasted_iota(jnp.int32, (cols, cols), 1)
    lt_incl = (ri <= ci).astype(jnp.float32)
    r2 = lax.broadcasted_iota(jnp.int32, (rows, rows), 0)
    c2 = lax.broadcasted_iota(jnp.int32, (rows, rows), 1)
    lt_excl = (c2 < r2).astype(jnp.float32)

    def mm(a, b):
        return lax.dot_general(a, b, (((1,), (0,)), ((), ())),
                               preferred_element_type=jnp.float32)

    pos = jnp.zeros((rows, cols), jnp.float32)
    off = jnp.zeros((1, 1), jnp.float32)
    ends = []
    for e in range(E):
        m = (eid == e).astype(jnp.float32)
        rc = mm(m, lt_incl)
        rt = jnp.sum(m, axis=1, keepdims=True)
        co = mm(lt_excl, rt)
        rank = co + rc - m
        pos = pos + m * (off + rank)
        cnt = jnp.sum(rt, axis=0, keepdims=True)
        off = off + jnp.ceil(cnt / T) * T
        ends.append(off)
    pos_ref[...] = pos.astype(jnp.int32)

    ends_cat = jnp.concatenate(ends, axis=1)
    bstart = (lax.broadcasted_iota(jnp.int32, (NB, 1), 0) * T).astype(jnp.float32)
    be = jnp.sum((ends_cat <= bstart).astype(jnp.int32), axis=1, keepdims=True)
    blk_ref[...] = jnp.where(bstart < off, jnp.minimum(be, E - 1), -1)


def _rank(eid2d):
    return pl.pallas_call(
        _rank_body,
        out_shape=(jax.ShapeDtypeStruct((N // 128, 128), jnp.int32),
                   jax.ShapeDtypeStruct((NB, 1), jnp.int32)),
    )(eid2d)


def _make_sc_dispatch():
    mesh = plsc.VectorSubcoreMesh(core_axis_name="c", subcore_axis_name="s",
                                  num_cores=2, num_subcores=16)

    @functools.partial(
        pl.kernel,
        out_type=jax.ShapeDtypeStruct((CAP, D), jnp.float32),
        mesh=mesh,
        scratch_types=[
            [pltpu.VMEM((CHUNK,), jnp.int32) for _ in range(NCH)],
            pltpu.VMEM((CHUNK, D), jnp.float32),
            pltpu.SemaphoreType.DMA,
        ],
    )
    def dispatch(pos_hbm, x_hbm, xs_hbm, idxs, rows_v, sem):
        wid = lax.axis_index("s") * 2 + lax.axis_index("c")
        base = wid * TPW
        for k in range(NCH):
            cb = base + k * CHUNK
            pltpu.sync_copy(pos_hbm.at[pl.ds(cb, CHUNK)], idxs[k])
            pltpu.sync_copy(x_hbm.at[pl.ds(cb, CHUNK)], rows_v)
            pltpu.async_copy(rows_v, xs_hbm.at[idxs[k]], sem).wait()

    return dispatch


def _expert_body(be_ref, xs_ref, w1_ref, b1_ref, w2_ref, b2_ref, ys_ref):
    e = be_ref[pl.program_id(0)]

    @pl.when(e >= 0)
    def _():
        w1 = w1_ref[e]
        w2 = w2_ref[e]
        b1 = b1_ref[e]
        b2 = b2_ref[e]
        ST = T // 4
        for s in range(T // ST):
            xb = xs_ref[pl.ds(s * ST, ST), :]
            h = jnp.dot(xb.astype(jnp.bfloat16), w1,
                        preferred_element_type=jnp.float32) + b1
            g = h * 0.5 * (1.0 + lax.erf(h * (1.0 / math.sqrt(2.0))))
            y = jnp.dot(g.astype(jnp.bfloat16), w2,
                        preferred_element_type=jnp.float32)
            ys_ref[pl.ds(s * ST, ST), :] = xb + y + b2


def _experts(blk_exp, xs, W1, b1, W2, b2):
    grid_spec = pltpu.PrefetchScalarGridSpec(
        num_scalar_prefetch=1,
        grid=(NB,),
        in_specs=[
            pl.BlockSpec((T, D), lambda i, be: (i, 0)),
            pl.BlockSpec((E, D, H), lambda i, be: (0, 0, 0)),
            pl.BlockSpec((E, 1, H), lambda i, be: (0, 0, 0)),
            pl.BlockSpec((E, H, D), lambda i, be: (0, 0, 0)),
            pl.BlockSpec((E, 1, D), lambda i, be: (0, 0, 0)),
        ],
        out_specs=pl.BlockSpec((T, D), lambda i, be: (i, 0)),
    )
    return pl.pallas_call(
        _expert_body,
        grid_spec=grid_spec,
        out_shape=jax.ShapeDtypeStruct((CAP, D), jnp.float32),
    )(blk_exp, xs, W1.astype(jnp.bfloat16), b1.reshape(E, 1, H),
      W2.astype(jnp.bfloat16), b2.reshape(E, 1, D))


def _make_sc_combine():
    mesh = plsc.VectorSubcoreMesh(core_axis_name="c", subcore_axis_name="s",
                                  num_cores=2, num_subcores=16)

    @functools.partial(
        pl.kernel,
        out_type=jax.ShapeDtypeStruct((N, D), jnp.float32),
        mesh=mesh,
        scratch_types=[
            [pltpu.VMEM((CHUNK,), jnp.int32) for _ in range(NCH)],
            pltpu.VMEM((CHUNK, D), jnp.float32),
            pltpu.SemaphoreType.DMA,
        ],
    )
    def combine(pos_hbm, ys_hbm, out_hbm, idxs, rows_v, sem):
        wid = lax.axis_index("s") * 2 + lax.axis_index("c")
        base = wid * TPW
        for k in range(NCH):
            cb = base + k * CHUNK
            pltpu.sync_copy(pos_hbm.at[pl.ds(cb, CHUNK)], idxs[k])
            pltpu.async_copy(ys_hbm.at[idxs[k]], rows_v, sem).wait()
            pltpu.sync_copy(rows_v, out_hbm.at[pl.ds(cb, CHUNK)])

    return combine


_sc_dispatch = functools.cache(_make_sc_dispatch)
_sc_combine = functools.cache(_make_sc_combine)


def kernel(x, Wr, br, W1, b1, W2, b2):
    B, S, _ = x.shape
    xf = x.reshape(N, D)
    eid = _router(xf, Wr, br)
    pos2d, blk = _rank(eid)
    pos = pos2d.reshape(N)
    xs = _sc_dispatch()(pos, xf)
    ys = _experts(blk.reshape(NB), xs, W1, b1, W2, b2)
    outf = _sc_combine()(pos, ys)
    return outf.reshape(B, S, D)
